# baseline (speedup 1.0000x reference)
import jax
import jax.numpy as jnp
from jax.experimental import pallas as pl
from jax.experimental.pallas import tpu as pltpu

E = 8
TOPK = 2
D = 1024
DFF = 512
T = 2048
NCH = 2
CH = T // NCH


def _body(x_ref, rw_ref, cb_ref, wgu_ref, wd_ref, out_ref, xb_ref, comb_ref):
    e = pl.program_id(0)

    @pl.when(e == 0)
    def _router():
        x = x_ref[...]
        xb_ref[...] = x.astype(jnp.bfloat16)
        logits = jnp.dot(x, rw_ref[...], preferred_element_type=jnp.float32)
        m = jnp.max(logits, axis=-1, keepdims=True)
        ex = jnp.exp(logits - m)
        scores = ex / jnp.sum(ex, axis=-1, keepdims=True)
        b = scores + cb_ref[...]
        ids = jax.lax.broadcasted_iota(jnp.int32, (T, E), 1)
        m1 = jnp.max(b, axis=-1, keepdims=True)
        i1 = jnp.min(jnp.where(b == m1, ids, E), axis=-1, keepdims=True)
        b2 = jnp.where(ids == i1, -1e30, b)
        m2 = jnp.max(b2, axis=-1, keepdims=True)
        i2 = jnp.min(jnp.where(b2 == m2, ids, E), axis=-1, keepdims=True)
        w1 = jnp.sum(jnp.where(ids == i1, scores, 0.0), axis=-1, keepdims=True)
        w2 = jnp.sum(jnp.where(ids == i2, scores, 0.0), axis=-1, keepdims=True)
        comb_ref[...] = jnp.where(ids == i1, w1, 0.0) + jnp.where(ids == i2, w2, 0.0)

    wgu = wgu_ref[0].astype(jnp.bfloat16)
    wd = wd_ref[0].astype(jnp.bfloat16)
    cslice = comb_ref[...]
    c_all = jnp.zeros((T, 1), jnp.float32)
    for j in range(E):
        c_all = c_all + jnp.where(e == j, cslice[:, j:j + 1], 0.0)

    for k in range(NCH):
        r0 = k * CH
        x = xb_ref[pl.ds(r0, CH), :]
        gu = jnp.dot(x, wgu, preferred_element_type=jnp.float32)
        gate = gu[:, :DFF]
        up = gu[:, DFF:]
        c = c_all[r0:r0 + CH, :]
        hw = (gate * jax.lax.logistic(gate) * up * c).astype(jnp.bfloat16)
        y = jnp.dot(hw, wd, preferred_element_type=jnp.float32)

        @pl.when(e == 0)
        def _init(y=y, r0=r0):
            out_ref[pl.ds(r0, CH), :] = y

        @pl.when(e != 0)
        def _acc(y=y, r0=r0):
            out_ref[pl.ds(r0, CH), :] += y


def kernel(hidden_states, router_w, correction_bias, w_gate_up, w_down):
    cb2 = correction_bias.reshape(1, E)
    out = pl.pallas_call(
        _body,
        grid=(E,),
        in_specs=[
            pl.BlockSpec((T, D), lambda e: (0, 0)),
            pl.BlockSpec((D, E), lambda e: (0, 0)),
            pl.BlockSpec((1, E), lambda e: (0, 0)),
            pl.BlockSpec((1, D, 2 * DFF), lambda e: (e, 0, 0)),
            pl.BlockSpec((1, DFF, D), lambda e: (e, 0, 0)),
        ],
        out_specs=pl.BlockSpec((T, D), lambda e: (0, 0)),
        out_shape=jax.ShapeDtypeStruct((T, D), jnp.float32),
        scratch_shapes=[
            pltpu.VMEM((T, D), jnp.bfloat16),
            pltpu.VMEM((T, E), jnp.float32),
        ],
    )(hidden_states, router_w, cb2, w_gate_up, w_down)
    return out


# speedup vs baseline: 1.0490x; 1.0490x over previous
import jax
import jax.numpy as jnp
from jax.experimental import pallas as pl
from jax.experimental.pallas import tpu as pltpu

E = 8
TOPK = 2
D = 1024
DFF = 512
T = 2048
NRC = 4
RC = T // NRC


def _router_body(x_ref, rw_ref, cb_ref, comb_ref, xb_ref):
    x = x_ref[...]
    xb_ref[...] = x.astype(jnp.bfloat16)
    logits = jnp.dot(x, rw_ref[...], preferred_element_type=jnp.float32)
    m = jnp.max(logits, axis=-1, keepdims=True)
    ex = jnp.exp(logits - m)
    scores = ex / jnp.sum(ex, axis=-1, keepdims=True)
    b = scores + cb_ref[...]
    ids = jax.lax.broadcasted_iota(jnp.int32, (T, E), 1)
    m1 = jnp.max(b, axis=-1, keepdims=True)
    i1 = jnp.min(jnp.where(b == m1, ids, E), axis=-1, keepdims=True)
    b2 = jnp.where(ids == i1, -1e30, b)
    m2 = jnp.max(b2, axis=-1, keepdims=True)
    i2 = jnp.min(jnp.where(b2 == m2, ids, E), axis=-1, keepdims=True)
    w1 = jnp.sum(jnp.where(ids == i1, scores, 0.0), axis=-1, keepdims=True)
    w2 = jnp.sum(jnp.where(ids == i2, scores, 0.0), axis=-1, keepdims=True)
    comb_ref[...] = jnp.where(ids == i1, w1, 0.0) + jnp.where(ids == i2, w2, 0.0)


def _moe_body(comb_ref, xb_ref, wgu_ref, wd_ref, out_ref, h_ref, wdb_ref):
    e = pl.program_id(0)

    @pl.when(e < E)
    def _expert():
        col0 = pl.multiple_of(e * DFF, DFF)
        wdb_ref[pl.ds(col0, DFF), :] = wd_ref[0].astype(jnp.bfloat16)
        wgu = wgu_ref[0].astype(jnp.bfloat16)
        gu = jnp.dot(xb_ref[...], wgu, preferred_element_type=jnp.float32)
        gate = gu[:, :DFF]
        up = gu[:, DFF:]
        h_ref[:, pl.ds(col0, DFF)] = (
            gate * jax.lax.logistic(gate) * up).astype(jnp.bfloat16)

    @pl.when(e == E)
    def _down():
        for j in range(E):
            cj = comb_ref[:, j:j + 1].astype(jnp.bfloat16)
            h_ref[:, j * DFF:(j + 1) * DFF] = h_ref[:, j * DFF:(j + 1) * DFF] * cj
        for r in range(NRC):
            r0 = r * RC
            out_ref[pl.ds(r0, RC), :] = jnp.dot(
                h_ref[pl.ds(r0, RC), :], wdb_ref[...],
                preferred_element_type=jnp.float32)


def kernel(hidden_states, router_w, correction_bias, w_gate_up, w_down):
    cb2 = correction_bias.reshape(1, E)
    comb, xb = pl.pallas_call(
        _router_body,
        out_shape=(
            jax.ShapeDtypeStruct((T, E), jnp.float32),
            jax.ShapeDtypeStruct((T, D), jnp.bfloat16),
        ),
    )(hidden_states, router_w, cb2)

    out = pl.pallas_call(
        _moe_body,
        grid=(E + 1,),
        in_specs=[
            pl.BlockSpec((T, E), lambda e: (0, 0)),
            pl.BlockSpec((T, D), lambda e: (0, 0)),
            pl.BlockSpec((1, D, 2 * DFF), lambda e: (jnp.minimum(e, E - 1), 0, 0)),
            pl.BlockSpec((1, DFF, D), lambda e: (jnp.minimum(e, E - 1), 0, 0)),
        ],
        out_specs=pl.BlockSpec((T, D), lambda e: (0, 0)),
        out_shape=jax.ShapeDtypeStruct((T, D), jnp.float32),
        scratch_shapes=[
            pltpu.VMEM((T, E * DFF), jnp.bfloat16),
            pltpu.VMEM((E * DFF, D), jnp.bfloat16),
        ],
        compiler_params=pltpu.CompilerParams(
            vmem_limit_bytes=64 * 1024 * 1024,
        ),
    )(comb, xb, w_gate_up, w_down)
    return out


# speedup vs baseline: 1.0508x; 1.0017x over previous
import jax
import jax.numpy as jnp
from jax.experimental import pallas as pl
from jax.experimental.pallas import tpu as pltpu

E = 8
TOPK = 2
D = 1024
DFF = 512
T = 2048
NRC = 2
RC = T // NRC


def _router_body(x_ref, rw_ref, cb_ref, comb_ref, xb_ref):
    x = x_ref[...]
    xb_ref[...] = x.astype(jnp.bfloat16)
    logits = jnp.dot(x, rw_ref[...], preferred_element_type=jnp.float32)
    m = jnp.max(logits, axis=-1, keepdims=True)
    ex = jnp.exp(logits - m)
    scores = ex / jnp.sum(ex, axis=-1, keepdims=True)
    b = scores + cb_ref[...]
    ids = jax.lax.broadcasted_iota(jnp.int32, (T, E), 1)
    m1 = jnp.max(b, axis=-1, keepdims=True)
    i1 = jnp.min(jnp.where(b == m1, ids, E), axis=-1, keepdims=True)
    b2 = jnp.where(ids == i1, -1e30, b)
    m2 = jnp.max(b2, axis=-1, keepdims=True)
    i2 = jnp.min(jnp.where(b2 == m2, ids, E), axis=-1, keepdims=True)
    w1 = jnp.sum(jnp.where(ids == i1, scores, 0.0), axis=-1, keepdims=True)
    w2 = jnp.sum(jnp.where(ids == i2, scores, 0.0), axis=-1, keepdims=True)
    comb_ref[...] = jnp.where(ids == i1, w1, 0.0) + jnp.where(ids == i2, w2, 0.0)


def _moe_body(comb_ref, xb_ref, wgu_ref, wd_ref, out_ref, h_ref, wdb_ref):
    e = pl.program_id(0)

    @pl.when(e < E)
    def _expert():
        col0 = pl.multiple_of(e * DFF, DFF)
        wdb_ref[pl.ds(col0, DFF), :] = wd_ref[0].astype(jnp.bfloat16)
        wgu = wgu_ref[0].astype(jnp.bfloat16)
        gu = jnp.dot(xb_ref[...], wgu, preferred_element_type=jnp.float32)
        gate = gu[:, :DFF]
        up = gu[:, DFF:]
        h_ref[:, pl.ds(col0, DFF)] = (
            gate * jax.lax.logistic(gate) * up).astype(jnp.bfloat16)

    @pl.when(e == E)
    def _down():
        for j in range(E):
            cj = comb_ref[:, j:j + 1].astype(jnp.bfloat16)
            h_ref[:, j * DFF:(j + 1) * DFF] = h_ref[:, j * DFF:(j + 1) * DFF] * cj
        for r in range(NRC):
            r0 = r * RC
            out_ref[pl.ds(r0, RC), :] = jnp.dot(
                h_ref[pl.ds(r0, RC), :], wdb_ref[...],
                preferred_element_type=jnp.float32)


def kernel(hidden_states, router_w, correction_bias, w_gate_up, w_down):
    cb2 = correction_bias.reshape(1, E)
    comb, xb = pl.pallas_call(
        _router_body,
        out_shape=(
            jax.ShapeDtypeStruct((T, E), jnp.float32),
            jax.ShapeDtypeStruct((T, D), jnp.bfloat16),
        ),
    )(hidden_states, router_w, cb2)

    out = pl.pallas_call(
        _moe_body,
        grid=(E + 1,),
        in_specs=[
            pl.BlockSpec((T, E), lambda e: (0, 0)),
            pl.BlockSpec((T, D), lambda e: (0, 0)),
            pl.BlockSpec((1, D, 2 * DFF), lambda e: (jnp.minimum(e, E - 1), 0, 0)),
            pl.BlockSpec((1, DFF, D), lambda e: (jnp.minimum(e, E - 1), 0, 0)),
        ],
        out_specs=pl.BlockSpec((T, D), lambda e: (0, 0)),
        out_shape=jax.ShapeDtypeStruct((T, D), jnp.float32),
        scratch_shapes=[
            pltpu.VMEM((T, E * DFF), jnp.bfloat16),
            pltpu.VMEM((E * DFF, D), jnp.bfloat16),
        ],
        compiler_params=pltpu.CompilerParams(
            vmem_limit_bytes=64 * 1024 * 1024,
        ),
    )(comb, xb, w_gate_up, w_down)
    return out


# speedup vs baseline: 1.0641x; 1.0126x over previous
import jax
import jax.numpy as jnp
from jax.experimental import pallas as pl
from jax.experimental.pallas import tpu as pltpu

E = 8
TOPK = 2
D = 1024
DFF = 512
T = 2048
NRC = 2
RC = T // NRC


def _router_body(x_ref, rw_ref, cb_ref, comb_ref, xb_ref):
    x = x_ref[...]
    xb_ref[...] = x.astype(jnp.bfloat16)
    logits = jnp.dot(x, rw_ref[...], preferred_element_type=jnp.float32)
    m = jnp.max(logits, axis=-1, keepdims=True)
    ex = jnp.exp(logits - m)
    scores = ex / jnp.sum(ex, axis=-1, keepdims=True)
    b = scores + cb_ref[...]
    ids = jax.lax.broadcasted_iota(jnp.int32, (T, E), 1)
    m1 = jnp.max(b, axis=-1, keepdims=True)
    i1 = jnp.min(jnp.where(b == m1, ids, E), axis=-1, keepdims=True)
    b2 = jnp.where(ids == i1, -1e30, b)
    m2 = jnp.max(b2, axis=-1, keepdims=True)
    i2 = jnp.min(jnp.where(b2 == m2, ids, E), axis=-1, keepdims=True)
    w1 = jnp.sum(jnp.where(ids == i1, scores, 0.0), axis=-1, keepdims=True)
    w2 = jnp.sum(jnp.where(ids == i2, scores, 0.0), axis=-1, keepdims=True)
    comb_ref[...] = jnp.where(ids == i1, w1, 0.0) + jnp.where(ids == i2, w2, 0.0)


def _moe_body(comb_ref, xb_ref, wgu_ref, wd_ref, out_ref, h_ref, wdb_ref):
    e = pl.program_id(0)

    @pl.when(e < E)
    def _expert():
        wdb_ref[e] = wd_ref[0].astype(jnp.bfloat16)
        wgu = wgu_ref[0].astype(jnp.bfloat16)
        gu = jnp.dot(xb_ref[...], wgu, preferred_element_type=jnp.float32)
        gate = gu[:, :DFF]
        up = gu[:, DFF:]
        h_ref[e] = (gate * jax.lax.logistic(gate) * up).astype(jnp.bfloat16)

    @pl.when(e == E)
    def _down():
        acc = jnp.zeros((T, D), jnp.float32)
        for j in range(E):
            cj = comb_ref[:, j:j + 1].astype(jnp.bfloat16)
            hw = h_ref[j] * cj
            acc = acc + jnp.dot(hw, wdb_ref[j],
                                preferred_element_type=jnp.float32)
        out_ref[...] = acc


def kernel(hidden_states, router_w, correction_bias, w_gate_up, w_down):
    cb2 = correction_bias.reshape(1, E)
    comb, xb = pl.pallas_call(
        _router_body,
        out_shape=(
            jax.ShapeDtypeStruct((T, E), jnp.float32),
            jax.ShapeDtypeStruct((T, D), jnp.bfloat16),
        ),
    )(hidden_states, router_w, cb2)

    out = pl.pallas_call(
        _moe_body,
        grid=(E + 1,),
        in_specs=[
            pl.BlockSpec((T, E), lambda e: (0, 0)),
            pl.BlockSpec((T, D), lambda e: (0, 0)),
            pl.BlockSpec((1, D, 2 * DFF), lambda e: (jnp.minimum(e, E - 1), 0, 0)),
            pl.BlockSpec((1, DFF, D), lambda e: (jnp.minimum(e, E - 1), 0, 0)),
        ],
        out_specs=pl.BlockSpec((T, D), lambda e: (0, 0)),
        out_shape=jax.ShapeDtypeStruct((T, D), jnp.float32),
        scratch_shapes=[
            pltpu.VMEM((E, T, DFF), jnp.bfloat16),
            pltpu.VMEM((E, DFF, D), jnp.bfloat16),
        ],
        compiler_params=pltpu.CompilerParams(
            vmem_limit_bytes=64 * 1024 * 1024,
        ),
    )(comb, xb, w_gate_up, w_down)
    return out
